# baseline (speedup 1.0000x reference)
import functools

import jax
import jax.numpy as jnp
from jax import lax
from jax.experimental import pallas as pl
from jax.experimental.pallas import tpu as pltpu
from jax.experimental.pallas import tpu_sc as plsc

N = 10000
K = 16
R = 200

NW = 32
RW = 320
NP = NW * RW
EW = RW * K
EP = NP * K


def _tc_topk_body(dst_ref, srcT_ref, idx_ref, dsti_ref):
    dblk = dst_ref[...]
    sT = srcT_ref[...]
    dn2 = jnp.sum(dblk * dblk, axis=1, keepdims=True)
    sn2 = jnp.sum(sT * sT, axis=0, keepdims=True)
    onesd = jnp.ones((R, 1), jnp.float32)
    oness = jnp.ones((1, N), jnp.float32)
    daug = jnp.concatenate([dblk * (-2.0), dn2, onesd], axis=1)
    saug = jnp.concatenate([sT, oness, sn2], axis=0)
    d2 = jnp.dot(daug, saug, preferred_element_type=jnp.float32)
    iota = jax.lax.broadcasted_iota(jnp.int32, (R, N), 1)
    bigi = jnp.int32(2**30)
    inf = jnp.float32(jnp.inf)
    idx_cols = []
    for _ in range(K):
        m = jnp.min(d2, axis=1, keepdims=True)
        sel = jnp.min(jnp.where(d2 == m, iota, bigi), axis=1, keepdims=True)
        d2 = jnp.where(iota == sel, inf, d2)
        idx_cols.append(sel)
    idx_ref[...] = jnp.concatenate(idx_cols, axis=1)
    row0 = pl.program_id(0) * R
    rows = row0 + jax.lax.broadcasted_iota(jnp.int32, (R, K), 0)
    dsti_ref[...] = rows


@functools.partial(jax.jit, static_argnames=("interpret",))
def _tc_topk(dst_pad, srcT_pad, interpret=False):
    return pl.pallas_call(
        _tc_topk_body,
        grid=(N // R,),
        in_specs=[
            pl.BlockSpec((R, 8), lambda i: (i, 0)),
            pl.BlockSpec((8, N), lambda i: (0, 0)),
        ],
        out_specs=[
            pl.BlockSpec((R, K), lambda i: (i, 0)),
            pl.BlockSpec((R, K), lambda i: (i, 0)),
        ],
        out_shape=[
            jax.ShapeDtypeStruct((N, K), jnp.int32),
            jax.ShapeDtypeStruct((N, K), jnp.int32),
        ],
        interpret=interpret,
    )(dst_pad, srcT_pad)


def _sc_attr_body(sx_hbm, sy_hbm, sz_hbm, dx_hbm, dy_hbm, dz_hbm, idx_hbm,
                  a0_hbm, a1_hbm, a2_hbm, a3_hbm,
                  sxv, syv, szv, dxv, dyv, dzv, idxv, a0, a1, a2, a3):
    c = lax.axis_index("c")
    s = lax.axis_index("s")
    w = s * 2 + c
    row0 = w * RW
    pltpu.sync_copy(sx_hbm, sxv)
    pltpu.sync_copy(sy_hbm, syv)
    pltpu.sync_copy(sz_hbm, szv)
    pltpu.sync_copy(dx_hbm, dxv)
    pltpu.sync_copy(dy_hbm, dyv)
    pltpu.sync_copy(dz_hbm, dzv)
    pltpu.sync_copy(idx_hbm.at[pl.ds(row0 * K, EW)], idxv)

    def body(r, carry):
        i = row0 + r
        si = idxv[pl.ds(r * K, K)]
        iv = jnp.full((K,), i, jnp.int32)
        dx = plsc.load_gather(dxv, [iv]) - plsc.load_gather(sxv, [si])
        dy = plsc.load_gather(dyv, [iv]) - plsc.load_gather(syv, [si])
        dz = plsc.load_gather(dzv, [iv]) - plsc.load_gather(szv, [si])
        d2 = dx * dx + dy * dy + dz * dz + jnp.float32(1e-12)
        yi = jnp.int32(0x5F3759DF) - lax.shift_right_logical(
            plsc.bitcast(d2, jnp.int32), 1)
        y = plsc.bitcast(yi, jnp.float32)
        h = jnp.float32(0.5) * d2
        for _ in range(3):
            y = y * (jnp.float32(1.5) - h * y * y)
        dist = d2 * y
        e0 = r * K
        a0[pl.ds(e0, K)] = dx
        a1[pl.ds(e0, K)] = dy
        a2[pl.ds(e0, K)] = dz
        a3[pl.ds(e0, K)] = dist
        return carry

    lax.fori_loop(0, RW, body, 0)
    pltpu.sync_copy(a0, a0_hbm.at[pl.ds(row0 * K, EW)])
    pltpu.sync_copy(a1, a1_hbm.at[pl.ds(row0 * K, EW)])
    pltpu.sync_copy(a2, a2_hbm.at[pl.ds(row0 * K, EW)])
    pltpu.sync_copy(a3, a3_hbm.at[pl.ds(row0 * K, EW)])


@functools.lru_cache(maxsize=1)
def _sc_attr_kernel():
  return pl.kernel(
    _sc_attr_body,
    out_type=[jax.ShapeDtypeStruct((EP,), jnp.float32)] * 4,
    mesh=plsc.VectorSubcoreMesh(core_axis_name="c", subcore_axis_name="s",
                                num_cores=2, num_subcores=16),
    compiler_params=pltpu.CompilerParams(needs_layout_passes=False),
    scratch_types=[
        pltpu.VMEM((N,), jnp.float32),
        pltpu.VMEM((N,), jnp.float32),
        pltpu.VMEM((N,), jnp.float32),
        pltpu.VMEM((NP,), jnp.float32),
        pltpu.VMEM((NP,), jnp.float32),
        pltpu.VMEM((NP,), jnp.float32),
        pltpu.VMEM((EW,), jnp.int32),
        pltpu.VMEM((EW,), jnp.float32),
        pltpu.VMEM((EW,), jnp.float32),
        pltpu.VMEM((EW,), jnp.float32),
        pltpu.VMEM((EW,), jnp.float32),
    ],
  )


def kernel(src_coords, dst_coords, interpret=False):
    dst_pad = jnp.pad(dst_coords, ((0, 0), (0, 5)))
    srcT_pad = jnp.pad(src_coords.T, ((0, 5), (0, 0)))
    idx, dsti = _tc_topk(dst_pad, srcT_pad, interpret=interpret)
    src_idx = idx.reshape(-1)
    dst_idx = dsti.reshape(-1)
    edge_index = jnp.stack([src_idx, dst_idx], axis=0)
    diff = jnp.take(dst_coords, dst_idx, axis=0) - jnp.take(src_coords, src_idx, axis=0)
    dist = jnp.sqrt(jnp.sum(diff * diff, axis=-1, keepdims=True) + 1e-12)
    edge_attr = jnp.concatenate([diff, dist], axis=-1)
    return edge_attr, edge_index


def _unused_sc_path(src_coords, dst_coords, idx):
    dpadT = jnp.pad(dst_coords.T, ((0, 0), (0, NP - N)))
    idx_flat = jnp.pad(idx, ((0, NP - N), (0, 0))).reshape(-1)
    a0, a1, a2, a3 = _sc_attr_kernel()(
        src_coords[:, 0], src_coords[:, 1], src_coords[:, 2],
        dpadT[0], dpadT[1], dpadT[2], idx_flat)
    edge_attr = jnp.stack(
        [a0[: N * K], a1[: N * K], a2[: N * K], a3[: N * K]], axis=1)
    return edge_attr


# speedup vs baseline: 1.0378x; 1.0378x over previous
import functools

import jax
import jax.numpy as jnp
from jax import lax
from jax.experimental import pallas as pl
from jax.experimental.pallas import tpu as pltpu
from jax.experimental.pallas import tpu_sc as plsc

N = 10000
K = 16
R = 200

NW = 32
RW = 320
NP = NW * RW
EW = RW * K
EP = NP * K


def _tc_topk_body(dst_ref, srcT_ref, idx_ref, dsti_ref):
    dblk = dst_ref[...]
    sT = srcT_ref[...]
    dn2 = jnp.sum(dblk * dblk, axis=1, keepdims=True)
    sn2 = jnp.sum(sT * sT, axis=0, keepdims=True)
    cross = jnp.dot(dblk, sT, preferred_element_type=jnp.float32)
    d2 = (dn2 - 2.0 * cross) + sn2
    iota = jax.lax.broadcasted_iota(jnp.int32, (R, N), 1)
    inf = jnp.float32(jnp.inf)
    idx_cols = []
    for _ in range(K):
        sel = jnp.argmin(d2, axis=1).astype(jnp.int32)[:, None]
        d2 = jnp.where(iota == sel, inf, d2)
        idx_cols.append(sel)
    idx_ref[...] = jnp.concatenate(idx_cols, axis=1)
    row0 = pl.program_id(0) * R
    rows = row0 + jax.lax.broadcasted_iota(jnp.int32, (R, K), 0)
    dsti_ref[...] = rows


@functools.partial(jax.jit, static_argnames=("interpret",))
def _tc_topk(dst_pad, srcT_pad, interpret=False):
    return pl.pallas_call(
        _tc_topk_body,
        grid=(N // R,),
        in_specs=[
            pl.BlockSpec((R, 8), lambda i: (i, 0)),
            pl.BlockSpec((8, N), lambda i: (0, 0)),
        ],
        out_specs=[
            pl.BlockSpec((R, K), lambda i: (i, 0)),
            pl.BlockSpec((R, K), lambda i: (i, 0)),
        ],
        out_shape=[
            jax.ShapeDtypeStruct((N, K), jnp.int32),
            jax.ShapeDtypeStruct((N, K), jnp.int32),
        ],
        interpret=interpret,
    )(dst_pad, srcT_pad)


def _sc_attr_body(sx_hbm, sy_hbm, sz_hbm, dx_hbm, dy_hbm, dz_hbm, idx_hbm,
                  a0_hbm, a1_hbm, a2_hbm, a3_hbm,
                  sxv, syv, szv, dxv, dyv, dzv, idxv, a0, a1, a2, a3):
    c = lax.axis_index("c")
    s = lax.axis_index("s")
    w = s * 2 + c
    row0 = w * RW
    pltpu.sync_copy(sx_hbm, sxv)
    pltpu.sync_copy(sy_hbm, syv)
    pltpu.sync_copy(sz_hbm, szv)
    pltpu.sync_copy(dx_hbm, dxv)
    pltpu.sync_copy(dy_hbm, dyv)
    pltpu.sync_copy(dz_hbm, dzv)
    pltpu.sync_copy(idx_hbm.at[pl.ds(row0 * K, EW)], idxv)

    def body(r, carry):
        i = row0 + r
        si = idxv[pl.ds(r * K, K)]
        iv = jnp.full((K,), i, jnp.int32)
        dx = plsc.load_gather(dxv, [iv]) - plsc.load_gather(sxv, [si])
        dy = plsc.load_gather(dyv, [iv]) - plsc.load_gather(syv, [si])
        dz = plsc.load_gather(dzv, [iv]) - plsc.load_gather(szv, [si])
        d2 = dx * dx + dy * dy + dz * dz + jnp.float32(1e-12)
        yi = jnp.int32(0x5F3759DF) - lax.shift_right_logical(
            plsc.bitcast(d2, jnp.int32), 1)
        y = plsc.bitcast(yi, jnp.float32)
        h = jnp.float32(0.5) * d2
        for _ in range(3):
            y = y * (jnp.float32(1.5) - h * y * y)
        dist = d2 * y
        e0 = r * K
        a0[pl.ds(e0, K)] = dx
        a1[pl.ds(e0, K)] = dy
        a2[pl.ds(e0, K)] = dz
        a3[pl.ds(e0, K)] = dist
        return carry

    lax.fori_loop(0, RW, body, 0)
    pltpu.sync_copy(a0, a0_hbm.at[pl.ds(row0 * K, EW)])
    pltpu.sync_copy(a1, a1_hbm.at[pl.ds(row0 * K, EW)])
    pltpu.sync_copy(a2, a2_hbm.at[pl.ds(row0 * K, EW)])
    pltpu.sync_copy(a3, a3_hbm.at[pl.ds(row0 * K, EW)])


@functools.lru_cache(maxsize=1)
def _sc_attr_kernel():
  return pl.kernel(
    _sc_attr_body,
    out_type=[jax.ShapeDtypeStruct((EP,), jnp.float32)] * 4,
    mesh=plsc.VectorSubcoreMesh(core_axis_name="c", subcore_axis_name="s",
                                num_cores=2, num_subcores=16),
    compiler_params=pltpu.CompilerParams(needs_layout_passes=False),
    scratch_types=[
        pltpu.VMEM((N,), jnp.float32),
        pltpu.VMEM((N,), jnp.float32),
        pltpu.VMEM((N,), jnp.float32),
        pltpu.VMEM((NP,), jnp.float32),
        pltpu.VMEM((NP,), jnp.float32),
        pltpu.VMEM((NP,), jnp.float32),
        pltpu.VMEM((EW,), jnp.int32),
        pltpu.VMEM((EW,), jnp.float32),
        pltpu.VMEM((EW,), jnp.float32),
        pltpu.VMEM((EW,), jnp.float32),
        pltpu.VMEM((EW,), jnp.float32),
    ],
  )


def kernel(src_coords, dst_coords, interpret=False):
    dst_pad = jnp.pad(dst_coords, ((0, 0), (0, 5)))
    srcT_pad = jnp.pad(src_coords.T, ((0, 5), (0, 0)))
    idx, dsti = _tc_topk(dst_pad, srcT_pad, interpret=interpret)
    src_idx = idx.reshape(-1)
    dst_idx = dsti.reshape(-1)
    edge_index = jnp.stack([src_idx, dst_idx], axis=0)
    diff = jnp.take(dst_coords, dst_idx, axis=0) - jnp.take(src_coords, src_idx, axis=0)
    dist = jnp.sqrt(jnp.sum(diff * diff, axis=-1, keepdims=True) + 1e-12)
    edge_attr = jnp.concatenate([diff, dist], axis=-1)
    return edge_attr, edge_index


def _unused_sc_path(src_coords, dst_coords, idx):
    dpadT = jnp.pad(dst_coords.T, ((0, 0), (0, NP - N)))
    idx_flat = jnp.pad(idx, ((0, NP - N), (0, 0))).reshape(-1)
    a0, a1, a2, a3 = _sc_attr_kernel()(
        src_coords[:, 0], src_coords[:, 1], src_coords[:, 2],
        dpadT[0], dpadT[1], dpadT[2], idx_flat)
    edge_attr = jnp.stack(
        [a0[: N * K], a1[: N * K], a2[: N * K], a3[: N * K]], axis=1)
    return edge_attr


# speedup vs baseline: 1.5688x; 1.5117x over previous
import functools

import jax
import jax.numpy as jnp
from jax import lax
from jax.experimental import pallas as pl
from jax.experimental.pallas import tpu as pltpu
from jax.experimental.pallas import tpu_sc as plsc

N = 10000
K = 16
R = 200

NW = 32
RW = 320
NP = NW * RW
EW = RW * K
EP = NP * K


def _tc_topk_body(dst_ref, srcT_ref, idx_ref, dsti_ref):
    dblk = dst_ref[...]
    sT = srcT_ref[...]
    dn2 = jnp.sum(dblk * dblk, axis=1, keepdims=True)
    sn2 = jnp.sum(sT * sT, axis=0, keepdims=True)
    cross = jnp.dot(dblk, sT, preferred_element_type=jnp.float32)
    d2 = (dn2 - 2.0 * cross) + sn2
    iota = jax.lax.broadcasted_iota(jnp.int32, (R, N), 1)
    inf = jnp.float32(jnp.inf)
    idx_cols = []
    for _ in range(K):
        sel = jnp.argmin(d2, axis=1).astype(jnp.int32)[:, None]
        d2 = jnp.where(iota == sel, inf, d2)
        idx_cols.append(sel)
    idx_ref[...] = jnp.concatenate(idx_cols, axis=1)
    row0 = pl.program_id(0) * R
    rows = row0 + jax.lax.broadcasted_iota(jnp.int32, (R, K), 0)
    dsti_ref[...] = rows


@functools.partial(jax.jit, static_argnames=("interpret",))
def _tc_topk(dst_pad, srcT_pad, interpret=False):
    return pl.pallas_call(
        _tc_topk_body,
        grid=(N // R,),
        in_specs=[
            pl.BlockSpec((R, 8), lambda i: (i, 0)),
            pl.BlockSpec((8, N), lambda i: (0, 0)),
        ],
        out_specs=[
            pl.BlockSpec((R, K), lambda i: (i, 0)),
            pl.BlockSpec((R, K), lambda i: (i, 0)),
        ],
        out_shape=[
            jax.ShapeDtypeStruct((N, K), jnp.int32),
            jax.ShapeDtypeStruct((N, K), jnp.int32),
        ],
        interpret=interpret,
    )(dst_pad, srcT_pad)


def _sc_attr_body(sx_hbm, sy_hbm, sz_hbm, dx_hbm, dy_hbm, dz_hbm, idx_hbm,
                  a0_hbm, a1_hbm, a2_hbm, a3_hbm,
                  sxv, syv, szv, dxv, dyv, dzv, idxv, a0, a1, a2, a3):
    c = lax.axis_index("c")
    s = lax.axis_index("s")
    w = s * 2 + c
    row0 = w * RW
    pltpu.sync_copy(sx_hbm, sxv)
    pltpu.sync_copy(sy_hbm, syv)
    pltpu.sync_copy(sz_hbm, szv)
    pltpu.sync_copy(dx_hbm, dxv)
    pltpu.sync_copy(dy_hbm, dyv)
    pltpu.sync_copy(dz_hbm, dzv)
    pltpu.sync_copy(idx_hbm.at[pl.ds(row0 * K, EW)], idxv)

    def body(r, carry):
        i = row0 + r
        si = idxv[pl.ds(r * K, K)]
        iv = jnp.full((K,), i, jnp.int32)
        dx = plsc.load_gather(dxv, [iv]) - plsc.load_gather(sxv, [si])
        dy = plsc.load_gather(dyv, [iv]) - plsc.load_gather(syv, [si])
        dz = plsc.load_gather(dzv, [iv]) - plsc.load_gather(szv, [si])
        d2 = dx * dx + dy * dy + dz * dz + jnp.float32(1e-12)
        yi = jnp.int32(0x5F3759DF) - lax.shift_right_logical(
            plsc.bitcast(d2, jnp.int32), 1)
        y = plsc.bitcast(yi, jnp.float32)
        h = jnp.float32(0.5) * d2
        for _ in range(3):
            y = y * (jnp.float32(1.5) - h * y * y)
        dist = d2 * y
        e0 = r * K
        a0[pl.ds(e0, K)] = dx
        a1[pl.ds(e0, K)] = dy
        a2[pl.ds(e0, K)] = dz
        a3[pl.ds(e0, K)] = dist
        return carry

    lax.fori_loop(0, RW, body, 0)
    pltpu.sync_copy(a0, a0_hbm.at[pl.ds(row0 * K, EW)])
    pltpu.sync_copy(a1, a1_hbm.at[pl.ds(row0 * K, EW)])
    pltpu.sync_copy(a2, a2_hbm.at[pl.ds(row0 * K, EW)])
    pltpu.sync_copy(a3, a3_hbm.at[pl.ds(row0 * K, EW)])


@functools.lru_cache(maxsize=1)
def _sc_attr_kernel():
  return pl.kernel(
    _sc_attr_body,
    out_type=[jax.ShapeDtypeStruct((EP,), jnp.float32)] * 4,
    mesh=plsc.VectorSubcoreMesh(core_axis_name="c", subcore_axis_name="s",
                                num_cores=2, num_subcores=16),
    compiler_params=pltpu.CompilerParams(needs_layout_passes=False),
    scratch_types=[
        pltpu.VMEM((N,), jnp.float32),
        pltpu.VMEM((N,), jnp.float32),
        pltpu.VMEM((N,), jnp.float32),
        pltpu.VMEM((NP,), jnp.float32),
        pltpu.VMEM((NP,), jnp.float32),
        pltpu.VMEM((NP,), jnp.float32),
        pltpu.VMEM((EW,), jnp.int32),
        pltpu.VMEM((EW,), jnp.float32),
        pltpu.VMEM((EW,), jnp.float32),
        pltpu.VMEM((EW,), jnp.float32),
        pltpu.VMEM((EW,), jnp.float32),
    ],
  )


def kernel(src_coords, dst_coords, interpret=False):
    dst_pad = jnp.pad(dst_coords, ((0, 0), (0, 5)))
    srcT_pad = jnp.pad(src_coords.T, ((0, 5), (0, 0)))
    idx, dsti = _tc_topk(dst_pad, srcT_pad, interpret=interpret)
    src_idx = idx.reshape(-1)
    dst_idx = dsti.reshape(-1)
    edge_index = jnp.stack([src_idx, dst_idx], axis=0)
    dpadT = jnp.pad(dst_coords.T, ((0, 0), (0, NP - N)))
    idx_flat = jnp.pad(idx, ((0, NP - N), (0, 0))).reshape(-1)
    a0, a1, a2, a3 = _sc_attr_kernel()(
        src_coords[:, 0], src_coords[:, 1], src_coords[:, 2],
        dpadT[0], dpadT[1], dpadT[2], idx_flat)
    edge_attr = jnp.stack(
        [a0[: N * K], a1[: N * K], a2[: N * K], a3[: N * K]], axis=1)
    return edge_attr, edge_index
